# baseline (speedup 1.0000x reference)
import functools

import jax
import jax.numpy as jnp
from jax import lax
from jax.experimental import pallas as pl
from jax.experimental.pallas import tpu as pltpu
from jax.experimental.pallas import tpu_sc as plsc

N = 10000
D = 128
EM = 128
E = 320000

NC = 2
NS = 16
L = 16
H = EM // NC

C = 80
CPS = E // NS
NB = 5
NSUP = CPS // (NB * C)
NIDX = 3
RPS_A = 624
BASE_LAST = (NS - 1) * RPS_A
RPS_LAST = N - BASE_LAST

_MM_BLK = 1000


def _row_slab_copy(get_src, get_dst, sid):
    base = pl.multiple_of(sid * RPS_A, 8)

    @pl.when(sid < NS - 1)
    def _():
        pltpu.sync_copy(get_src(base, RPS_A), get_dst(base, RPS_A))

    @pl.when(sid == NS - 1)
    def _():
        pltpu.sync_copy(get_src(BASE_LAST, RPS_LAST),
                        get_dst(BASE_LAST, RPS_LAST))


def _spmm_body(x0_hbm, x1_hbm, col_hbm, row_hbm, vals_hbm, out_hbm,
               colg, rowg, valsg, rows_g, rows_s, acc, gsems, ssems, isem):
    cid = lax.axis_index("c")
    sid = lax.axis_index("s")

    def phase(xc_hbm, col0):
        _row_slab_copy(lambda b, n: xc_hbm.at[pl.ds(b, n)],
                       lambda b, n: acc.at[pl.ds(b, n)], sid)

        pltpu.sync_copy(col_hbm.at[sid, 0], colg.at[0])
        pltpu.sync_copy(row_hbm.at[sid, 0], rowg.at[0])
        pltpu.sync_copy(vals_hbm.at[sid, 0], valsg.at[0])
        plsc.subcore_barrier()
        for b in range(NB):
            pltpu.async_copy(xc_hbm.at[colg.at[0, b]], rows_g.at[b],
                             gsems.at[b])

        @pl.loop(0, NSUP)
        def _super(s):
            p = lax.rem(s, NIDX)
            pm1 = lax.rem(s + (NIDX - 1), NIDX)
            pp1 = lax.rem(s + 1, NIDX)

            @pl.when(s < NSUP - 1)
            def _():
                pltpu.async_copy(col_hbm.at[sid, s + 1], colg.at[pp1], isem)
                pltpu.async_copy(row_hbm.at[sid, s + 1], rowg.at[pp1], isem)
                pltpu.async_copy(vals_hbm.at[sid, s + 1], valsg.at[pp1],
                                 isem)

            for b in range(NB):
                pltpu.make_async_copy(xc_hbm.at[colg.at[p, b]],
                                      rows_g.at[b], gsems.at[b]).wait()

                @pl.when(s > 0)
                def _():
                    pltpu.make_async_copy(rows_s.at[b],
                                          acc.at[rowg.at[pm1, b]],
                                          ssems.at[b]).wait()

                @pl.loop(0, C // L, unroll=5)
                def _grp(g):
                    vv = valsg[p, b, pl.ds(g * L, L)]
                    for t in range(L):
                        v = vv[t]
                        kk = g * L + t
                        for j in range(H // L):
                            sl = pl.ds(j * L, L)
                            rows_s[b, kk, sl] = rows_g[b, kk, sl] * v

                pltpu.async_copy(rows_s.at[b], acc.at[rowg.at[p, b]],
                                 ssems.at[b], add=True)

                @pl.when(s < NSUP - 1)
                def _():
                    if b == 0:
                        pltpu.make_async_copy(col_hbm.at[sid, s + 1],
                                              colg.at[pp1], isem).wait()
                        pltpu.make_async_copy(row_hbm.at[sid, s + 1],
                                              rowg.at[pp1], isem).wait()
                        pltpu.make_async_copy(vals_hbm.at[sid, s + 1],
                                              valsg.at[pp1], isem).wait()
                    pltpu.async_copy(xc_hbm.at[colg.at[pp1, b]],
                                     rows_g.at[b], gsems.at[b])

        for b in range(NB):
            pltpu.make_async_copy(rows_s.at[b],
                                  acc.at[rowg.at[(NSUP - 1) % NIDX, b]],
                                  ssems.at[b]).wait()
        plsc.subcore_barrier()
        _row_slab_copy(
            lambda b, n: acc.at[pl.ds(b, n)],
            lambda b, n: out_hbm.at[pl.ds(b, n), pl.ds(col0, H)], sid)

    @pl.when(cid == 0)
    def _():
        phase(x0_hbm, 0)

    @pl.when(cid == 1)
    def _():
        phase(x1_hbm, H)


@functools.partial(
    pl.kernel,
    out_type=jax.ShapeDtypeStruct((N, EM), jnp.float32),
    mesh=plsc.VectorSubcoreMesh(core_axis_name="c", subcore_axis_name="s"),
    compiler_params=pltpu.CompilerParams(use_tc_tiling_on_sc=False),
    scratch_types=[
        pltpu.VMEM((NIDX, NB, C), jnp.int32),
        pltpu.VMEM((NIDX, NB, C), jnp.int32),
        pltpu.VMEM((NIDX, NB, C), jnp.float32),
        pltpu.VMEM((NB, C, H), jnp.float32),
        pltpu.VMEM((NB, C, H), jnp.float32),
        pltpu.VMEM_SHARED((N, H), jnp.float32),
        pltpu.SemaphoreType.DMA((NB,)),
        pltpu.SemaphoreType.DMA((NB,)),
        pltpu.SemaphoreType.DMA,
    ],
)
def _spmm_sc(x0_hbm, x1_hbm, col_hbm, row_hbm, vals_hbm, out_hbm,
             colg, rowg, valsg, rows_g, rows_s, acc, gsems, ssems, isem):
    _spmm_body(x0_hbm, x1_hbm, col_hbm, row_hbm, vals_hbm, out_hbm,
               colg, rowg, valsg, rows_g, rows_s, acc, gsems, ssems, isem)


def _mm_body(x_ref, w_ref, o0_ref, o1_ref):
    r = jnp.dot(x_ref[...], w_ref[...], preferred_element_type=jnp.float32)
    o0_ref[...] = r[:, :H]
    o1_ref[...] = r[:, H:]


def _matmul_split(x, w):
    grid = x.shape[0] // _MM_BLK
    return pl.pallas_call(
        _mm_body,
        grid=(grid,),
        in_specs=[
            pl.BlockSpec((_MM_BLK, x.shape[1]), lambda i: (i, 0)),
            pl.BlockSpec(w.shape, lambda i: (0, 0)),
        ],
        out_specs=[
            pl.BlockSpec((_MM_BLK, H), lambda i: (i, 0)),
            pl.BlockSpec((_MM_BLK, H), lambda i: (i, 0)),
        ],
        out_shape=[
            jax.ShapeDtypeStruct((x.shape[0], H), jnp.float32),
            jax.ShapeDtypeStruct((x.shape[0], H), jnp.float32),
        ],
    )(x, w)


def _bn_mm_body(h_ref, g_ref, b_ref, w_ref, o0_ref, o1_ref, accs_ref):
    ph = pl.program_id(0)
    i = pl.program_id(1)

    @pl.when((ph == 0) & (i == 0))
    def _():
        accs_ref[...] = jnp.zeros_like(accs_ref)

    @pl.when(ph == 0)
    def _():
        h = h_ref[...]
        s = jnp.sum(h, axis=0, keepdims=True)
        sq = jnp.sum(h * h, axis=0, keepdims=True)
        accs_ref[0:2, :] += jnp.concatenate([s, sq], axis=0)

        @pl.when(i == pl.num_programs(1) - 1)
        def _():
            mean = accs_ref[0:1, :] / N
            var = accs_ref[1:2, :] / N - mean * mean
            inv = lax.rsqrt(var + 1e-5)
            scale = g_ref[...] * inv
            shift = b_ref[...] - mean * scale
            accs_ref[2:4, :] = jnp.concatenate([scale, shift], axis=0)

    @pl.when(ph == 1)
    def _():
        scale = accs_ref[2:3, :]
        shift = accs_ref[3:4, :]
        hn = jnp.maximum(h_ref[...] * scale + shift, 0.0)
        r = jnp.dot(hn, w_ref[...], preferred_element_type=jnp.float32)
        o0_ref[...] = r[:, :H]
        o1_ref[...] = r[:, H:]


def _bn_mm_split(h, gamma, beta, w):
    grid = (2, N // _MM_BLK)
    return pl.pallas_call(
        _bn_mm_body,
        grid=grid,
        in_specs=[
            pl.BlockSpec((_MM_BLK, EM), lambda ph, i: (i, 0)),
            pl.BlockSpec((1, EM), lambda ph, i: (0, 0)),
            pl.BlockSpec((1, EM), lambda ph, i: (0, 0)),
            pl.BlockSpec((EM, EM), lambda ph, i: (0, 0)),
        ],
        out_specs=[
            pl.BlockSpec((_MM_BLK, H), lambda ph, i: (i, 0)),
            pl.BlockSpec((_MM_BLK, H), lambda ph, i: (i, 0)),
        ],
        out_shape=[
            jax.ShapeDtypeStruct((N, H), jnp.float32),
            jax.ShapeDtypeStruct((N, H), jnp.float32),
        ],
        scratch_shapes=[pltpu.VMEM((4, EM), jnp.float32)],
    )(h, gamma.reshape(1, EM), beta.reshape(1, EM), w)


def kernel(feature, adj_indices, adj_values, W1, W2, bn_gamma, bn_beta):
    row = adj_indices[0]
    col = adj_indices[1]

    col_r = col.reshape(NS, NSUP, NB, C)
    row_r = row.reshape(NS, NSUP, NB, C)
    vals_r = adj_values.reshape(NS, NSUP, NB, C)

    s0, s1 = _matmul_split(feature, W1)
    h = _spmm_sc(s0, s1, col_r, row_r, vals_r)
    h20, h21 = _bn_mm_split(h, bn_gamma, bn_beta, W2)
    return _spmm_sc(h20, h21, col_r, row_r, vals_r)


# speedup vs baseline: 1.2907x; 1.2907x over previous
import functools

import jax
import jax.numpy as jnp
from jax import lax
from jax.experimental import pallas as pl
from jax.experimental.pallas import tpu as pltpu
from jax.experimental.pallas import tpu_sc as plsc

N = 10000
D = 128
EM = 128
E = 320000

NC = 2
NS = 16
L = 16
H = EM // NC

C = 80
CPS = E // NS
NB = 5
NSUP = CPS // (NB * C)
NIDX = 3
RPS_A = 624
BASE_LAST = (NS - 1) * RPS_A
RPS_LAST = N - BASE_LAST

_MM_BLK = 1000


def _row_slab_copy(get_src, get_dst, sid):
    base = pl.multiple_of(sid * RPS_A, 8)

    @pl.when(sid < NS - 1)
    def _():
        pltpu.sync_copy(get_src(base, RPS_A), get_dst(base, RPS_A))

    @pl.when(sid == NS - 1)
    def _():
        pltpu.sync_copy(get_src(BASE_LAST, RPS_LAST),
                        get_dst(BASE_LAST, RPS_LAST))


def _spmm_body(x0_hbm, x1_hbm, col_hbm, row_hbm, vals_hbm, out_hbm,
               colg, rowg, valsg, rows_g, rows_s, acc, gsems, ssems, isem):
    cid = lax.axis_index("c")
    sid = lax.axis_index("s")

    def phase(xc_hbm, col0):
        _row_slab_copy(lambda b, n: xc_hbm.at[pl.ds(b, n)],
                       lambda b, n: acc.at[pl.ds(b, n)], sid)

        pltpu.sync_copy(col_hbm.at[sid, 0], colg.at[0])
        pltpu.sync_copy(row_hbm.at[sid, 0], rowg.at[0])
        pltpu.sync_copy(vals_hbm.at[sid, 0], valsg.at[0])
        plsc.subcore_barrier()
        for b in range(NB):
            pltpu.async_copy(xc_hbm.at[colg.at[0, b]], rows_g.at[b],
                             gsems.at[b])

        @pl.loop(0, NSUP)
        def _super(s):
            p = lax.rem(s, NIDX)
            pm1 = lax.rem(s + (NIDX - 1), NIDX)
            pp1 = lax.rem(s + 1, NIDX)

            @pl.when(s < NSUP - 1)
            def _():
                pltpu.async_copy(col_hbm.at[sid, s + 1], colg.at[pp1], isem)
                pltpu.async_copy(row_hbm.at[sid, s + 1], rowg.at[pp1], isem)
                pltpu.async_copy(vals_hbm.at[sid, s + 1], valsg.at[pp1],
                                 isem)

            for b in range(NB):
                pltpu.make_async_copy(xc_hbm.at[colg.at[p, b]],
                                      rows_g.at[b], gsems.at[b]).wait()

                @pl.when(s > 0)
                def _():
                    pltpu.make_async_copy(rows_s.at[b],
                                          acc.at[rowg.at[pm1, b]],
                                          ssems.at[b]).wait()

                @pl.loop(0, C // L)
                def _grp(g):
                    vv = valsg[p, b, pl.ds(g * L, L)]
                    for t in range(L):
                        v = vv[t]
                        kk = g * L + t
                        for j in range(H // L):
                            sl = pl.ds(j * L, L)
                            rows_s[b, kk, sl] = rows_g[b, kk, sl] * v

                pltpu.async_copy(rows_s.at[b], acc.at[rowg.at[p, b]],
                                 ssems.at[b], add=True)

                @pl.when(s < NSUP - 1)
                def _():
                    if b == 0:
                        pltpu.make_async_copy(col_hbm.at[sid, s + 1],
                                              colg.at[pp1], isem).wait()
                        pltpu.make_async_copy(row_hbm.at[sid, s + 1],
                                              rowg.at[pp1], isem).wait()
                        pltpu.make_async_copy(vals_hbm.at[sid, s + 1],
                                              valsg.at[pp1], isem).wait()
                    pltpu.async_copy(xc_hbm.at[colg.at[pp1, b]],
                                     rows_g.at[b], gsems.at[b])

        for b in range(NB):
            pltpu.make_async_copy(rows_s.at[b],
                                  acc.at[rowg.at[(NSUP - 1) % NIDX, b]],
                                  ssems.at[b]).wait()
        plsc.subcore_barrier()
        _row_slab_copy(
            lambda b, n: acc.at[pl.ds(b, n)],
            lambda b, n: out_hbm.at[pl.ds(b, n), pl.ds(col0, H)], sid)

    @pl.when(cid == 0)
    def _():
        phase(x0_hbm, 0)

    @pl.when(cid == 1)
    def _():
        phase(x1_hbm, H)


@functools.partial(
    pl.kernel,
    out_type=jax.ShapeDtypeStruct((N, EM), jnp.float32),
    mesh=plsc.VectorSubcoreMesh(core_axis_name="c", subcore_axis_name="s"),
    compiler_params=pltpu.CompilerParams(use_tc_tiling_on_sc=False),
    scratch_types=[
        pltpu.VMEM((NIDX, NB, C), jnp.int32),
        pltpu.VMEM((NIDX, NB, C), jnp.int32),
        pltpu.VMEM((NIDX, NB, C), jnp.float32),
        pltpu.VMEM((NB, C, H), jnp.float32),
        pltpu.VMEM((NB, C, H), jnp.float32),
        pltpu.VMEM_SHARED((N, H), jnp.float32),
        pltpu.SemaphoreType.DMA((NB,)),
        pltpu.SemaphoreType.DMA((NB,)),
        pltpu.SemaphoreType.DMA,
    ],
)
def _spmm_sc(x0_hbm, x1_hbm, col_hbm, row_hbm, vals_hbm, out_hbm,
             colg, rowg, valsg, rows_g, rows_s, acc, gsems, ssems, isem):
    _spmm_body(x0_hbm, x1_hbm, col_hbm, row_hbm, vals_hbm, out_hbm,
               colg, rowg, valsg, rows_g, rows_s, acc, gsems, ssems, isem)


def _mm_body(x_ref, w_ref, o0_ref, o1_ref):
    r = jnp.dot(x_ref[...], w_ref[...], preferred_element_type=jnp.float32)
    o0_ref[...] = r[:, :H]
    o1_ref[...] = r[:, H:]


def _matmul_split(x, w):
    grid = x.shape[0] // _MM_BLK
    return pl.pallas_call(
        _mm_body,
        grid=(grid,),
        in_specs=[
            pl.BlockSpec((_MM_BLK, x.shape[1]), lambda i: (i, 0)),
            pl.BlockSpec(w.shape, lambda i: (0, 0)),
        ],
        out_specs=[
            pl.BlockSpec((_MM_BLK, H), lambda i: (i, 0)),
            pl.BlockSpec((_MM_BLK, H), lambda i: (i, 0)),
        ],
        out_shape=[
            jax.ShapeDtypeStruct((x.shape[0], H), jnp.float32),
            jax.ShapeDtypeStruct((x.shape[0], H), jnp.float32),
        ],
    )(x, w)


def _bn_mm_body(h_ref, g_ref, b_ref, w_ref, o0_ref, o1_ref, accs_ref):
    ph = pl.program_id(0)
    i = pl.program_id(1)

    @pl.when((ph == 0) & (i == 0))
    def _():
        accs_ref[...] = jnp.zeros_like(accs_ref)

    @pl.when(ph == 0)
    def _():
        h = h_ref[...]
        s = jnp.sum(h, axis=0, keepdims=True)
        sq = jnp.sum(h * h, axis=0, keepdims=True)
        accs_ref[0:2, :] += jnp.concatenate([s, sq], axis=0)

        @pl.when(i == pl.num_programs(1) - 1)
        def _():
            mean = accs_ref[0:1, :] / N
            var = accs_ref[1:2, :] / N - mean * mean
            inv = lax.rsqrt(var + 1e-5)
            scale = g_ref[...] * inv
            shift = b_ref[...] - mean * scale
            accs_ref[2:4, :] = jnp.concatenate([scale, shift], axis=0)

    @pl.when(ph == 1)
    def _():
        scale = accs_ref[2:3, :]
        shift = accs_ref[3:4, :]
        hn = jnp.maximum(h_ref[...] * scale + shift, 0.0)
        r = jnp.dot(hn, w_ref[...], preferred_element_type=jnp.float32)
        o0_ref[...] = r[:, :H]
        o1_ref[...] = r[:, H:]


def _bn_mm_split(h, gamma, beta, w):
    grid = (2, N // _MM_BLK)
    return pl.pallas_call(
        _bn_mm_body,
        grid=grid,
        in_specs=[
            pl.BlockSpec((_MM_BLK, EM), lambda ph, i: (i, 0)),
            pl.BlockSpec((1, EM), lambda ph, i: (0, 0)),
            pl.BlockSpec((1, EM), lambda ph, i: (0, 0)),
            pl.BlockSpec((EM, EM), lambda ph, i: (0, 0)),
        ],
        out_specs=[
            pl.BlockSpec((_MM_BLK, H), lambda ph, i: (i, 0)),
            pl.BlockSpec((_MM_BLK, H), lambda ph, i: (i, 0)),
        ],
        out_shape=[
            jax.ShapeDtypeStruct((N, H), jnp.float32),
            jax.ShapeDtypeStruct((N, H), jnp.float32),
        ],
        scratch_shapes=[pltpu.VMEM((4, EM), jnp.float32)],
    )(h, gamma.reshape(1, EM), beta.reshape(1, EM), w)


def kernel(feature, adj_indices, adj_values, W1, W2, bn_gamma, bn_beta):
    row = adj_indices[0]
    col = adj_indices[1]

    col_r = col.reshape(NS, NSUP, NB, C)
    row_r = row.reshape(NS, NSUP, NB, C)
    vals_r = adj_values.reshape(NS, NSUP, NB, C)

    s0, s1 = _matmul_split(feature, W1)
    h = _spmm_sc(s0, s1, col_r, row_r, vals_r)
    h20, h21 = _bn_mm_split(h, bn_gamma, bn_beta, W2)
    return _spmm_sc(h20, h21, col_r, row_r, vals_r)
